# baseline (speedup 1.0000x reference)
import functools

import jax
import jax.numpy as jnp
from jax import lax
from jax.experimental import pallas as pl
from jax.experimental.pallas import tpu as pltpu
from jax.experimental.pallas import tpu_sc as plsc

N = 10000
E = 320000
H = 128
G = 64
L = 3

NC = 2
NS = 16
NW = NC * NS

CH = 128
NGRP = 39
CHPW = 2 * NGRP
EPW = CHPW * CH
EPW2 = EPW // 2
WPC = CH // 2
TAIL0 = NW * EPW
NTAIL = (E - TAIL0) // CH
ZFULL = N // CH
ZREM = N - ZFULL * CH

BN = 1000
NBLK = N // BN


_sc_mesh = plsc.VectorSubcoreMesh(core_axis_name="c", subcore_axis_name="s",
                                  num_cores=NC, num_subcores=NS)


@functools.partial(
    pl.kernel,
    out_type=jax.ShapeDtypeStruct((NC, N, H), jnp.float32),
    mesh=_sc_mesh,
    scratch_types=[
        pltpu.VMEM((EPW2,), jnp.int32),
        pltpu.VMEM((EPW2,), jnp.int32),
        [pltpu.VMEM((CH,), jnp.int32) for _ in range(2)],
        [pltpu.VMEM((CH,), jnp.int32) for _ in range(2)],
        [pltpu.VMEM((CH, H), jnp.float32) for _ in range(2)],
        pltpu.VMEM_SHARED((N, H), jnp.float32),
        pltpu.SemaphoreType.DMA,
        pltpu.SemaphoreType.DMA,
        pltpu.SemaphoreType.DMA,
        pltpu.SemaphoreType.DMA,
        pltpu.SemaphoreType.DMA,
    ],
)
def _segsum_sc(h_hbm, spk_hbm, dpk_hbm, out_hbm, srcp_v, dstp_v, src_vs,
               dst_vs, rows_vs, agg_sp, sem_g, sem_i, sem_s, sem_g2, sem_s2):
    c = lax.axis_index("c")
    s = lax.axis_index("s")
    wid = c * NS + s

    d_si = pltpu.async_copy(spk_hbm.at[pl.ds(wid * EPW2, EPW2)], srcp_v,
                            sem_i)
    d_di = pltpu.async_copy(dpk_hbm.at[pl.ds(wid * EPW2, EPW2)], dstp_v,
                            sem_i)

    def zfill(i, _):
        rows_vs[0][i // 8, pl.ds((i % 8) * 16, 16)] = jnp.zeros((16,),
                                                               jnp.float32)
        return 0
    lax.fori_loop(0, CH * 8, zfill, 0)

    def zblk(k, _):
        b = s + k * NS

        @pl.when(b < ZFULL)
        def _():
            pltpu.sync_copy(rows_vs[0], agg_sp.at[pl.ds(b * CH, CH)])

        @pl.when(b == ZFULL)
        def _():
            pltpu.sync_copy(rows_vs[0].at[pl.ds(0, ZREM)],
                            agg_sp.at[pl.ds(ZFULL * CH, ZREM)])
        return 0
    lax.fori_loop(0, -(-(ZFULL + 1) // NS), zblk, 0)

    d_si.wait()
    d_di.wait()
    plsc.subcore_barrier()

    sem_gs = [sem_g, sem_g2]
    sem_ss = [sem_s, sem_s2]

    def expand_idx(j, p):
        jo = pl.multiple_of(j * WPC, 16)
        for k in range(CH // 32):
            w = srcp_v[pl.ds(jo + k * 16, 16)]
            src_vs[p][pl.ds(k * 32, 16)] = w & 0xFFFF
            src_vs[p][pl.ds(k * 32 + 16, 16)] = w >> 16
            w = dstp_v[pl.ds(jo + k * 16, 16)]
            dst_vs[p][pl.ds(k * 32, 16)] = w & 0xFFFF
            dst_vs[p][pl.ds(k * 32 + 16, 16)] = w >> 16

    def issue_gather(j, p):
        expand_idx(j, p)
        pltpu.async_copy(h_hbm.at[src_vs[p]], rows_vs[p], sem_gs[p])

    def drain_gather(p):
        pltpu.make_async_copy(h_hbm.at[src_vs[p]],
                              rows_vs[p], sem_gs[p]).wait()

    def drain_scatter(p):
        pltpu.make_async_copy(rows_vs[p],
                              agg_sp.at[dst_vs[p]], sem_ss[p]).wait()

    issue_gather(0, 0)

    def ebody(g, _):
        for b in range(2):
            j = g * 2 + b
            q = 1 - b
            drain_gather(b)

            @pl.when(j >= 1)
            def _():
                drain_scatter(q)

            @pl.when(j + 1 < CHPW)
            def _():
                issue_gather(j + 1, q)
            pltpu.async_copy(rows_vs[b], agg_sp.at[dst_vs[b]], sem_ss[b],
                             add=True)
        return 0
    lax.fori_loop(0, NGRP, ebody, 0)
    drain_scatter(1)

    @pl.when(wid < NTAIL)
    def _():
        tb2 = TAIL0 // 2 + wid * WPC
        pltpu.sync_copy(spk_hbm.at[pl.ds(tb2, WPC)], srcp_v.at[pl.ds(0, WPC)])
        pltpu.sync_copy(dpk_hbm.at[pl.ds(tb2, WPC)], dstp_v.at[pl.ds(0, WPC)])
        expand_idx(0, 0)
        pltpu.async_copy(h_hbm.at[src_vs[0]], rows_vs[0], sem_g).wait()
        pltpu.sync_copy(rows_vs[0], agg_sp.at[dst_vs[0]], add=True)

    plsc.subcore_barrier()

    def wblk(k, _):
        b = s + k * NS

        @pl.when(b < ZFULL)
        def _():
            pltpu.sync_copy(agg_sp.at[pl.ds(b * CH, CH)],
                            out_hbm.at[c, pl.ds(b * CH, CH)])

        @pl.when(b == ZFULL)
        def _():
            pltpu.sync_copy(agg_sp.at[pl.ds(ZFULL * CH, ZREM)],
                            out_hbm.at[c, pl.ds(ZFULL * CH, ZREM)])
        return 0
    lax.fori_loop(0, -(-(ZFULL + 1) // NS), wblk, 0)


def _mlp_body(ep_ref, h_ref, p0_ref, p1_ref, w1_ref, b1_ref, w2_ref, b2_ref,
              sc_ref, sb_ref, out_ref):
    ep = ep_ref[0]
    z = h_ref[...] * ep + p0_ref[...] + p1_ref[...]
    z = jnp.maximum(jnp.dot(z, w1_ref[...],
                            preferred_element_type=jnp.float32) + b1_ref[...], 0.0)
    z = jnp.maximum(jnp.dot(z, w2_ref[...],
                            preferred_element_type=jnp.float32) + b2_ref[...], 0.0)
    out_ref[...] = z * sc_ref[...] + sb_ref[...]


def _mlp_tc(h, p0, p1, w1, b1, w2, b2, scale, shift, epsp1):
    return pl.pallas_call(
        _mlp_body,
        grid=(NBLK,),
        in_specs=[
            pl.BlockSpec(memory_space=pltpu.SMEM),
            pl.BlockSpec((BN, H), lambda i: (i, 0)),
            pl.BlockSpec((BN, H), lambda i: (i, 0)),
            pl.BlockSpec((BN, H), lambda i: (i, 0)),
            pl.BlockSpec((H, H), lambda i: (0, 0)),
            pl.BlockSpec((1, H), lambda i: (0, 0)),
            pl.BlockSpec((H, H), lambda i: (0, 0)),
            pl.BlockSpec((1, H), lambda i: (0, 0)),
            pl.BlockSpec((1, H), lambda i: (0, 0)),
            pl.BlockSpec((1, H), lambda i: (0, 0)),
        ],
        out_specs=pl.BlockSpec((BN, H), lambda i: (i, 0)),
        out_shape=jax.ShapeDtypeStruct((N, H), jnp.float32),
    )(epsp1, h, p0, p1, w1, b1, w2, b2, scale, shift)


def _final_body(ep_ref, b_ref, h1_ref, h2_ref, p0_ref, p1_ref, w1_ref,
                b1_ref, w2_ref, b2_ref, sc_ref, sb_ref, l1w_ref, l1b_ref,
                l2w_ref, l2b_ref, logp_ref, em_ref, acc, cnt):
    i = pl.program_id(0)

    @pl.when(i == 0)
    def _():
        acc[...] = jnp.zeros_like(acc)
        cnt[...] = jnp.zeros_like(cnt)

    z = h2_ref[...] * ep_ref[0] + p0_ref[...] + p1_ref[...]
    z = jnp.maximum(jnp.dot(z, w1_ref[...],
                            preferred_element_type=jnp.float32) + b1_ref[...], 0.0)
    z = jnp.maximum(jnp.dot(z, w2_ref[...],
                            preferred_element_type=jnp.float32) + b2_ref[...], 0.0)
    h3 = z * sc_ref[...] + sb_ref[...]

    em_ref[:, 0:H] = h1_ref[...]
    em_ref[:, H:2 * H] = h2_ref[...]
    em_ref[:, 2 * H:3 * H] = h3

    b = b_ref[0, 0, :]
    iota_g = lax.broadcasted_iota(jnp.int32, (BN, G), 1)
    onehot = (b[:, None] == iota_g).astype(jnp.float32)
    dn = (((0,), (0,)), ((), ()))
    acc[:, 0:H] += lax.dot_general(onehot, h1_ref[...], dn,
                                   preferred_element_type=jnp.float32)
    acc[:, H:2 * H] += lax.dot_general(onehot, h2_ref[...], dn,
                                       preferred_element_type=jnp.float32)
    acc[:, 2 * H:3 * H] += lax.dot_general(onehot, h3, dn,
                                           preferred_element_type=jnp.float32)
    cnt[...] += jnp.sum(onehot, axis=0)[:, None]

    @pl.when(i == NBLK - 1)
    def _():
        inv = 1.0 / jnp.maximum(cnt[...], 1.0)
        h2a = l1b_ref[...]
        for j in range(L):
            pooled = acc[:, j * H:(j + 1) * H] * inv
            h2a = h2a + jnp.dot(pooled, l1w_ref[pl.ds(j * H, H), :],
                                preferred_element_type=jnp.float32)
        h2a = jnp.maximum(h2a, 0.0)
        logits = jnp.dot(h2a, l2w_ref[...],
                         preferred_element_type=jnp.float32) + l2b_ref[...]
        m = jnp.max(logits, axis=-1, keepdims=True)
        lse = jnp.log(jnp.sum(jnp.exp(logits - m), axis=-1, keepdims=True)) + m
        logp_ref[...] = logits - lse


def _final_tc(batch_r, h1, h2, p0, p1, w1, b1, w2, b2, scale, shift, epsp1,
              l1w, l1b, l2w, l2b):
    return pl.pallas_call(
        _final_body,
        grid=(NBLK,),
        in_specs=[
            pl.BlockSpec(memory_space=pltpu.SMEM),
            pl.BlockSpec((1, 1, BN), lambda i: (i, 0, 0)),
            pl.BlockSpec((BN, H), lambda i: (i, 0)),
            pl.BlockSpec((BN, H), lambda i: (i, 0)),
            pl.BlockSpec((BN, H), lambda i: (i, 0)),
            pl.BlockSpec((BN, H), lambda i: (i, 0)),
            pl.BlockSpec((H, H), lambda i: (0, 0)),
            pl.BlockSpec((1, H), lambda i: (0, 0)),
            pl.BlockSpec((H, H), lambda i: (0, 0)),
            pl.BlockSpec((1, H), lambda i: (0, 0)),
            pl.BlockSpec((1, H), lambda i: (0, 0)),
            pl.BlockSpec((1, H), lambda i: (0, 0)),
            pl.BlockSpec((L * H, H), lambda i: (0, 0)),
            pl.BlockSpec((1, H), lambda i: (0, 0)),
            pl.BlockSpec((H, H), lambda i: (0, 0)),
            pl.BlockSpec((1, H), lambda i: (0, 0)),
        ],
        out_specs=[
            pl.BlockSpec((G, H), lambda i: (0, 0)),
            pl.BlockSpec((BN, L * H), lambda i: (i, 0)),
        ],
        out_shape=[
            jax.ShapeDtypeStruct((G, H), jnp.float32),
            jax.ShapeDtypeStruct((N, L * H), jnp.float32),
        ],
        scratch_shapes=[
            pltpu.VMEM((G, L * H), jnp.float32),
            pltpu.VMEM((G, H), jnp.float32),
        ],
    )(epsp1, batch_r, h1, h2, p0, p1, w1, b1, w2, b2, scale, shift,
      l1w, l1b, l2w, l2b)


@jax.jit
def kernel(x, edge_index, batch, W1, b1, W2, b2, bnw, bnb, eps,
           lin1_W, lin1_b, lin2_W, lin2_b):
    spk = lax.bitcast_convert_type(
        edge_index[0].astype(jnp.int16).reshape(E // 2, 2), jnp.int32)
    dpk = lax.bitcast_convert_type(
        edge_index[1].astype(jnp.int16).reshape(E // 2, 2), jnp.int32)
    batch_r = batch.astype(jnp.int32).reshape(NBLK, 1, BN)

    bn_scale = (bnw / jnp.sqrt(1.0 + 1e-5)).reshape(L, 1, H)
    bn_shift = bnb.reshape(L, 1, H)
    b1r = b1.reshape(L, 1, H)
    b2r = b2.reshape(L, 1, H)
    epsp1 = (1.0 + eps).reshape(L, 1)

    h = x
    hs = []
    for i in range(L - 1):
        parts = _segsum_sc(h, spk, dpk)
        h = _mlp_tc(h, parts[0], parts[1], W1[i], b1r[i], W2[i], b2r[i],
                    bn_scale[i], bn_shift[i], epsp1[i])
        hs.append(h)

    parts = _segsum_sc(h, spk, dpk)
    logp, em = _final_tc(batch_r, hs[0], hs[1], parts[0], parts[1],
                         W1[2], b1r[2], W2[2], b2r[2], bn_scale[2],
                         bn_shift[2], epsp1[2],
                         lin1_W, lin1_b.reshape(1, H), lin2_W,
                         lin2_b.reshape(1, H))
    return (logp, em)


# speedup vs baseline: 1.6005x; 1.6005x over previous
import functools

import jax
import jax.numpy as jnp
from jax import lax
from jax.experimental import pallas as pl
from jax.experimental.pallas import tpu as pltpu
from jax.experimental.pallas import tpu_sc as plsc

N = 10000
E = 320000
H = 128
G = 64
L = 3

NC = 2
NS = 16
NW = NC * NS

CH = 128
GRP = 2
NGRP = 39
CHPW = GRP * NGRP
EPW = CHPW * CH
TAIL0 = NW * EPW
NTAIL = (E - TAIL0) // CH
ZFULL = N // CH
ZREM = N - ZFULL * CH

BN = 1000
NBLK = N // BN


_sc_mesh = plsc.VectorSubcoreMesh(core_axis_name="c", subcore_axis_name="s",
                                  num_cores=NC, num_subcores=NS)


@functools.partial(
    pl.kernel,
    out_type=jax.ShapeDtypeStruct((NC, N, H), jnp.float32),
    mesh=_sc_mesh,
    scratch_types=[
        pltpu.VMEM((EPW,), jnp.int32),
        [pltpu.VMEM((CH,), jnp.int32) for _ in range(GRP)],
        [pltpu.VMEM((CH, H), jnp.float32) for _ in range(GRP)],
        pltpu.VMEM_SHARED((N, H), jnp.float32),
        pltpu.SemaphoreType.DMA,
        pltpu.SemaphoreType.DMA,
        pltpu.SemaphoreType.DMA,
        pltpu.SemaphoreType.DMA,
        pltpu.SemaphoreType.DMA,
    ],
)
def _segsum_sc(h_hbm, src_hbm, dst_hbm, out_hbm, srcall_v, dst_vs, rows_vs,
               agg_sp, sem_g, sem_i, sem_s, sem_g2, sem_s2):
    c = lax.axis_index("c")
    s = lax.axis_index("s")
    wid = c * NS + s
    base_e = wid * EPW

    d_idx = pltpu.async_copy(src_hbm.at[pl.ds(base_e, EPW)], srcall_v, sem_i)

    def zfill(i, _):
        rows_vs[1][i // 8, pl.ds((i % 8) * 16, 16)] = jnp.zeros((16,),
                                                               jnp.float32)
        return 0
    lax.fori_loop(0, CH * 8, zfill, 0)

    d_idx.wait()

    sem_gs = [sem_g, sem_g2]
    sem_ss = [sem_s, sem_s2]

    def issue_gather(j, p):
        pltpu.async_copy(dst_hbm.at[pl.ds(base_e + j * CH, CH)],
                         dst_vs[p], sem_gs[p])
        pltpu.async_copy(h_hbm.at[srcall_v.at[pl.ds(j * CH, CH)]],
                         rows_vs[p], sem_gs[p])

    def drain_gather(p):
        pltpu.make_async_copy(dst_hbm.at[pl.ds(base_e, CH)],
                              dst_vs[p], sem_gs[p]).wait()
        pltpu.make_async_copy(h_hbm.at[srcall_v.at[pl.ds(0, CH)]],
                              rows_vs[p], sem_gs[p]).wait()

    def drain_scatter(p):
        pltpu.make_async_copy(rows_vs[p],
                              agg_sp.at[dst_vs[p]], sem_ss[p]).wait()

    issue_gather(0, 0)

    def zblk(k, _):
        b = s + k * NS

        @pl.when(b < ZFULL)
        def _():
            pltpu.sync_copy(rows_vs[1], agg_sp.at[pl.ds(b * CH, CH)])

        @pl.when(b == ZFULL)
        def _():
            pltpu.sync_copy(rows_vs[1].at[pl.ds(0, ZREM)],
                            agg_sp.at[pl.ds(ZFULL * CH, ZREM)])
        return 0
    lax.fori_loop(0, -(-(ZFULL + 1) // NS), zblk, 0)

    plsc.subcore_barrier()

    def ebody(g, _):
        for b in range(2):
            j = g * 2 + b
            q = 1 - b
            drain_gather(b)

            @pl.when(j >= 1)
            def _():
                drain_scatter(q)

            @pl.when(j + 1 < CHPW)
            def _():
                issue_gather(j + 1, q)
            pltpu.async_copy(rows_vs[b], agg_sp.at[dst_vs[b]], sem_ss[b],
                             add=True)
        return 0
    lax.fori_loop(0, NGRP, ebody, 0)
    drain_scatter(1)

    @pl.when(wid < NTAIL)
    def _():
        tb = TAIL0 + wid * CH
        pltpu.sync_copy(src_hbm.at[pl.ds(tb, CH)], dst_vs[1])
        pltpu.sync_copy(dst_hbm.at[pl.ds(tb, CH)], dst_vs[0])
        pltpu.async_copy(h_hbm.at[dst_vs[1]], rows_vs[0], sem_g).wait()
        pltpu.sync_copy(rows_vs[0], agg_sp.at[dst_vs[0]], add=True)

    plsc.subcore_barrier()

    def wblk(k, _):
        b = s + k * NS

        @pl.when(b < ZFULL)
        def _():
            pltpu.sync_copy(agg_sp.at[pl.ds(b * CH, CH)],
                            out_hbm.at[c, pl.ds(b * CH, CH)])

        @pl.when(b == ZFULL)
        def _():
            pltpu.sync_copy(agg_sp.at[pl.ds(ZFULL * CH, ZREM)],
                            out_hbm.at[c, pl.ds(ZFULL * CH, ZREM)])
        return 0
    lax.fori_loop(0, -(-(ZFULL + 1) // NS), wblk, 0)


def _mlp_body(ep_ref, h_ref, p0_ref, p1_ref, w1_ref, b1_ref, w2_ref, b2_ref,
              sc_ref, sb_ref, out_ref):
    ep = ep_ref[0]
    z = h_ref[...] * ep + p0_ref[...] + p1_ref[...]
    z = jnp.maximum(jnp.dot(z, w1_ref[...],
                            preferred_element_type=jnp.float32) + b1_ref[...], 0.0)
    z = jnp.maximum(jnp.dot(z, w2_ref[...],
                            preferred_element_type=jnp.float32) + b2_ref[...], 0.0)
    out_ref[...] = z * sc_ref[...] + sb_ref[...]


def _mlp_tc(h, p0, p1, w1, b1, w2, b2, scale, shift, epsp1):
    return pl.pallas_call(
        _mlp_body,
        grid=(NBLK,),
        in_specs=[
            pl.BlockSpec(memory_space=pltpu.SMEM),
            pl.BlockSpec((BN, H), lambda i: (i, 0)),
            pl.BlockSpec((BN, H), lambda i: (i, 0)),
            pl.BlockSpec((BN, H), lambda i: (i, 0)),
            pl.BlockSpec((H, H), lambda i: (0, 0)),
            pl.BlockSpec((1, H), lambda i: (0, 0)),
            pl.BlockSpec((H, H), lambda i: (0, 0)),
            pl.BlockSpec((1, H), lambda i: (0, 0)),
            pl.BlockSpec((1, H), lambda i: (0, 0)),
            pl.BlockSpec((1, H), lambda i: (0, 0)),
        ],
        out_specs=pl.BlockSpec((BN, H), lambda i: (i, 0)),
        out_shape=jax.ShapeDtypeStruct((N, H), jnp.float32),
    )(epsp1, h, p0, p1, w1, b1, w2, b2, scale, shift)


def _final_body(ep_ref, b_ref, h1_ref, h2_ref, p0_ref, p1_ref, w1_ref,
                b1_ref, w2_ref, b2_ref, sc_ref, sb_ref, l1w_ref, l1b_ref,
                l2w_ref, l2b_ref, logp_ref, em_ref, acc, cnt):
    i = pl.program_id(0)

    @pl.when(i == 0)
    def _():
        acc[...] = jnp.zeros_like(acc)
        cnt[...] = jnp.zeros_like(cnt)

    z = h2_ref[...] * ep_ref[0] + p0_ref[...] + p1_ref[...]
    z = jnp.maximum(jnp.dot(z, w1_ref[...],
                            preferred_element_type=jnp.float32) + b1_ref[...], 0.0)
    z = jnp.maximum(jnp.dot(z, w2_ref[...],
                            preferred_element_type=jnp.float32) + b2_ref[...], 0.0)
    h3 = z * sc_ref[...] + sb_ref[...]

    em_ref[:, 0:H] = h1_ref[...]
    em_ref[:, H:2 * H] = h2_ref[...]
    em_ref[:, 2 * H:3 * H] = h3

    b = b_ref[0, 0, :]
    iota_g = lax.broadcasted_iota(jnp.int32, (BN, G), 1)
    onehot = (b[:, None] == iota_g).astype(jnp.float32)
    dn = (((0,), (0,)), ((), ()))
    acc[:, 0:H] += lax.dot_general(onehot, h1_ref[...], dn,
                                   preferred_element_type=jnp.float32)
    acc[:, H:2 * H] += lax.dot_general(onehot, h2_ref[...], dn,
                                       preferred_element_type=jnp.float32)
    acc[:, 2 * H:3 * H] += lax.dot_general(onehot, h3, dn,
                                           preferred_element_type=jnp.float32)
    cnt[...] += jnp.sum(onehot, axis=0)[:, None]

    @pl.when(i == NBLK - 1)
    def _():
        inv = 1.0 / jnp.maximum(cnt[...], 1.0)
        h2a = l1b_ref[...]
        for j in range(L):
            pooled = acc[:, j * H:(j + 1) * H] * inv
            h2a = h2a + jnp.dot(pooled, l1w_ref[pl.ds(j * H, H), :],
                                preferred_element_type=jnp.float32)
        h2a = jnp.maximum(h2a, 0.0)
        logits = jnp.dot(h2a, l2w_ref[...],
                         preferred_element_type=jnp.float32) + l2b_ref[...]
        m = jnp.max(logits, axis=-1, keepdims=True)
        lse = jnp.log(jnp.sum(jnp.exp(logits - m), axis=-1, keepdims=True)) + m
        logp_ref[...] = logits - lse


def _final_tc(batch_r, h1, h2, p0, p1, w1, b1, w2, b2, scale, shift, epsp1,
              l1w, l1b, l2w, l2b):
    return pl.pallas_call(
        _final_body,
        grid=(NBLK,),
        in_specs=[
            pl.BlockSpec(memory_space=pltpu.SMEM),
            pl.BlockSpec((1, 1, BN), lambda i: (i, 0, 0)),
            pl.BlockSpec((BN, H), lambda i: (i, 0)),
            pl.BlockSpec((BN, H), lambda i: (i, 0)),
            pl.BlockSpec((BN, H), lambda i: (i, 0)),
            pl.BlockSpec((BN, H), lambda i: (i, 0)),
            pl.BlockSpec((H, H), lambda i: (0, 0)),
            pl.BlockSpec((1, H), lambda i: (0, 0)),
            pl.BlockSpec((H, H), lambda i: (0, 0)),
            pl.BlockSpec((1, H), lambda i: (0, 0)),
            pl.BlockSpec((1, H), lambda i: (0, 0)),
            pl.BlockSpec((1, H), lambda i: (0, 0)),
            pl.BlockSpec((L * H, H), lambda i: (0, 0)),
            pl.BlockSpec((1, H), lambda i: (0, 0)),
            pl.BlockSpec((H, H), lambda i: (0, 0)),
            pl.BlockSpec((1, H), lambda i: (0, 0)),
        ],
        out_specs=[
            pl.BlockSpec((G, H), lambda i: (0, 0)),
            pl.BlockSpec((BN, L * H), lambda i: (i, 0)),
        ],
        out_shape=[
            jax.ShapeDtypeStruct((G, H), jnp.float32),
            jax.ShapeDtypeStruct((N, L * H), jnp.float32),
        ],
        scratch_shapes=[
            pltpu.VMEM((G, L * H), jnp.float32),
            pltpu.VMEM((G, H), jnp.float32),
        ],
    )(epsp1, batch_r, h1, h2, p0, p1, w1, b1, w2, b2, scale, shift,
      l1w, l1b, l2w, l2b)


@jax.jit
def kernel(x, edge_index, batch, W1, b1, W2, b2, bnw, bnb, eps,
           lin1_W, lin1_b, lin2_W, lin2_b):
    src = edge_index[0].astype(jnp.int32)
    dst = edge_index[1].astype(jnp.int32)
    batch_r = batch.astype(jnp.int32).reshape(NBLK, 1, BN)

    bn_scale = (bnw / jnp.sqrt(1.0 + 1e-5)).reshape(L, 1, H)
    bn_shift = bnb.reshape(L, 1, H)
    b1r = b1.reshape(L, 1, H)
    b2r = b2.reshape(L, 1, H)
    epsp1 = (1.0 + eps).reshape(L, 1)

    h = x
    hs = []
    for i in range(L - 1):
        parts = _segsum_sc(h, src, dst)
        h = _mlp_tc(h, parts[0], parts[1], W1[i], b1r[i], W2[i], b2r[i],
                    bn_scale[i], bn_shift[i], epsp1[i])
        hs.append(h)

    parts = _segsum_sc(h, src, dst)
    logp, em = _final_tc(batch_r, hs[0], hs[1], parts[0], parts[1],
                         W1[2], b1r[2], W2[2], b2r[2], bn_scale[2],
                         bn_shift[2], epsp1[2],
                         lin1_W, lin1_b.reshape(1, H), lin2_W,
                         lin2_b.reshape(1, H))
    return (logp, em)
